# baseline (speedup 1.0000x reference)
import functools

import jax
import jax.numpy as jnp
from jax import lax
from jax.experimental import pallas as pl
from jax.experimental.pallas import tpu as pltpu
from jax.experimental.pallas import tpu_sc as plsc

H = 64
NE = 32
NA = 16
NC = 2
NS = 16
NW = NC * NS
NB = 2


def _table_body(eemb_ref, aemb_ref, w_ref, b_ref, t_ref):
    we = w_ref[:, :H]
    wa = w_ref[:, H:]
    dn = (((1,), (1,)), ((), ()))
    pe = lax.dot_general(eemb_ref[...], we, dn,
                         preferred_element_type=jnp.float32,
                         precision=lax.Precision.HIGHEST)
    pa = lax.dot_general(aemb_ref[...], wa, dn,
                         preferred_element_type=jnp.float32,
                         precision=lax.Precision.HIGHEST)
    t = pe[:, None, :] + pa[None, :, :] + b_ref[0][None, None, :]
    t_ref[...] = jnp.concatenate([t, jnp.zeros_like(t)], axis=-1)


def _idx_body(e_ref, a_ref, o_ref):
    o_ref[...] = e_ref[...] * NA + a_ref[...]


def _make_sc_body(B, L):
    bpw = B // NW

    def _sc_gather_body(t_hbm, idx_hbm, out_hbm, idx_v, rows128, rows64,
                        *sems):
        sems_g, sems_s = sems[:NB], sems[NB:]
        wid = lax.axis_index("s") * NC + lax.axis_index("c")
        b_base = wid * bpw
        pltpu.sync_copy(idx_hbm.at[pl.ds(b_base * L, bpw * L)], idx_v)

        def gather_parts(j, s):
            return [
                pltpu.make_async_copy(
                    t_hbm.at[idx_v.at[pl.ds(j * L, 128)]],
                    rows128.at[s, pl.ds(0, 128)], sems_g[s]),
                pltpu.make_async_copy(
                    t_hbm.at[idx_v.at[pl.ds(j * L + 128, L - 128)]],
                    rows128.at[s, pl.ds(128, L - 128)], sems_g[s]),
            ]

        def store(j, sc):
            return pltpu.make_async_copy(rows64.at[sc],
                                         out_hbm.at[b_base + j], sems_s[sc])

        def compact(s):
            def row(r, _):
                for c in range(H // 16):
                    rows64[s, r, pl.ds(c * 16, 16)] = (
                        rows128[s, r, pl.ds(c * 16, 16)])
                return 0

            lax.fori_loop(0, L, row, 0)

        for s in range(NB):
            for g in gather_parts(s, s):
                g.start()

        n_rounds = bpw // NB

        def round_body(m, _):
            j0 = m * NB
            for s in range(NB):
                j = j0 + s
                for g in gather_parts(j, s):
                    g.wait()

                @pl.when(m > 0)
                def _():
                    store(j - NB, s).wait()

                compact(s)
                store(j, s).start()

                @pl.when(m + 1 < n_rounds)
                def _():
                    for g in gather_parts(j + NB, s):
                        g.start()
            return 0

        lax.fori_loop(0, n_rounds, round_body, 0)
        for s in range(NB):
            store(bpw - NB + s, s).wait()

    return _sc_gather_body


def kernel(entity, attr, entity_emb, attr_emb, W, b):
    B, L = entity.shape
    n = B * L
    assert B % NW == 0 and L == 200

    t3 = pl.pallas_call(
        _table_body,
        out_shape=jax.ShapeDtypeStruct((NE, NA, 2 * H), jnp.float32),
    )(entity_emb, attr_emb, W, b.reshape(1, H))
    table = t3.reshape(NE * NA, 2 * H)

    e2 = entity.reshape(n // 128, 128)
    a2 = attr.reshape(n // 128, 128)
    rows_per_blk = n // 128 // 8
    idx = pl.pallas_call(
        _idx_body,
        grid=(8,),
        in_specs=[pl.BlockSpec((rows_per_blk, 128), lambda i: (i, 0))] * 2,
        out_specs=pl.BlockSpec((rows_per_blk, 128), lambda i: (i, 0)),
        out_shape=jax.ShapeDtypeStruct((n // 128, 128), jnp.int32),
    )(e2, a2)

    bpw = B // NW
    mesh = plsc.VectorSubcoreMesh(core_axis_name="c", subcore_axis_name="s",
                                  num_cores=NC, num_subcores=NS)
    gather = functools.partial(
        pl.kernel,
        out_type=jax.ShapeDtypeStruct((B, L, H), jnp.float32),
        mesh=mesh,
        compiler_params=pltpu.CompilerParams(use_tc_tiling_on_sc=True),
        scratch_types=(
            [pltpu.VMEM((bpw * L,), jnp.int32),
             pltpu.VMEM((NB, L, 2 * H), jnp.float32),
             pltpu.VMEM((NB, L, H), jnp.float32)]
            + [pltpu.SemaphoreType.DMA] * (2 * NB)
        ),
    )(_make_sc_body(B, L))
    return gather(table, idx.reshape(n))


# speedup vs baseline: 1.0010x; 1.0010x over previous
import functools

import jax
import jax.numpy as jnp
from jax import lax
from jax.experimental import pallas as pl
from jax.experimental.pallas import tpu as pltpu
from jax.experimental.pallas import tpu_sc as plsc

H = 64
NE = 32
NA = 16
NC = 2
NS = 16
NW = NC * NS
NB = 2


def _table_body(eemb_ref, aemb_ref, w_ref, b_ref, t_ref):
    we = w_ref[:, :H]
    wa = w_ref[:, H:]
    dn = (((1,), (1,)), ((), ()))
    pe = lax.dot_general(eemb_ref[...], we, dn,
                         preferred_element_type=jnp.float32,
                         precision=lax.Precision.HIGHEST)
    pa = lax.dot_general(aemb_ref[...], wa, dn,
                         preferred_element_type=jnp.float32,
                         precision=lax.Precision.HIGHEST)
    t = pe[:, None, :] + pa[None, :, :] + b_ref[0][None, None, :]
    t_ref[...] = jnp.concatenate([t, jnp.zeros_like(t)], axis=-1)


def _idx_body(e_ref, a_ref, o_ref):
    o_ref[...] = e_ref[...] * NA + a_ref[...]


def _make_sc_body(B, L):
    bpw = B // NW

    def _sc_gather_body(t_hbm, idx_hbm, out_hbm, idx_v, rows128, rows64,
                        *sems):
        sems_g, sems_s = sems[:NB], sems[NB:]
        wid = lax.axis_index("s") * NC + lax.axis_index("c")
        b_base = wid * bpw
        pltpu.sync_copy(idx_hbm.at[pl.ds(b_base * L, bpw * L)], idx_v)

        def gather_parts(j, s):
            return [
                pltpu.make_async_copy(
                    t_hbm.at[idx_v.at[pl.ds(j * L, 128)]],
                    rows128.at[s, pl.ds(0, 128)], sems_g[s]),
                pltpu.make_async_copy(
                    t_hbm.at[idx_v.at[pl.ds(j * L + 128, L - 128)]],
                    rows128.at[s, pl.ds(128, L - 128)], sems_g[s]),
            ]

        def store(j, sc):
            return pltpu.make_async_copy(rows64.at[sc],
                                         out_hbm.at[b_base + j], sems_s[sc])

        def compact(s):
            def row(r4, _):
                for u in range(4):
                    r = r4 * 4 + u
                    for c in range(H // 16):
                        rows64[s, r, pl.ds(c * 16, 16)] = (
                            rows128[s, r, pl.ds(c * 16, 16)])
                return 0

            lax.fori_loop(0, L // 4, row, 0)

        for s in range(NB):
            for g in gather_parts(s, s):
                g.start()

        n_rounds = bpw // NB

        def round_body(m, _):
            j0 = m * NB
            for s in range(NB):
                j = j0 + s
                for g in gather_parts(j, s):
                    g.wait()

                @pl.when(m > 0)
                def _():
                    store(j - NB, s).wait()

                compact(s)
                store(j, s).start()

                @pl.when(m + 1 < n_rounds)
                def _():
                    for g in gather_parts(j + NB, s):
                        g.start()
            return 0

        lax.fori_loop(0, n_rounds, round_body, 0)
        for s in range(NB):
            store(bpw - NB + s, s).wait()

    return _sc_gather_body


def kernel(entity, attr, entity_emb, attr_emb, W, b):
    B, L = entity.shape
    n = B * L
    assert B % NW == 0 and L == 200

    t3 = pl.pallas_call(
        _table_body,
        out_shape=jax.ShapeDtypeStruct((NE, NA, 2 * H), jnp.float32),
    )(entity_emb, attr_emb, W, b.reshape(1, H))
    table = t3.reshape(NE * NA, 2 * H)

    e2 = entity.reshape(n // 128, 128)
    a2 = attr.reshape(n // 128, 128)
    rows_per_blk = n // 128 // 8
    idx = pl.pallas_call(
        _idx_body,
        grid=(8,),
        in_specs=[pl.BlockSpec((rows_per_blk, 128), lambda i: (i, 0))] * 2,
        out_specs=pl.BlockSpec((rows_per_blk, 128), lambda i: (i, 0)),
        out_shape=jax.ShapeDtypeStruct((n // 128, 128), jnp.int32),
    )(e2, a2)

    bpw = B // NW
    mesh = plsc.VectorSubcoreMesh(core_axis_name="c", subcore_axis_name="s",
                                  num_cores=NC, num_subcores=NS)
    gather = functools.partial(
        pl.kernel,
        out_type=jax.ShapeDtypeStruct((B, L, H), jnp.float32),
        mesh=mesh,
        compiler_params=pltpu.CompilerParams(use_tc_tiling_on_sc=True),
        scratch_types=(
            [pltpu.VMEM((bpw * L,), jnp.int32),
             pltpu.VMEM((NB, L, 2 * H), jnp.float32),
             pltpu.VMEM((NB, L, H), jnp.float32)]
            + [pltpu.SemaphoreType.DMA] * (2 * NB)
        ),
    )(_make_sc_body(B, L))
    return gather(table, idx.reshape(n))


# speedup vs baseline: 1.1150x; 1.1139x over previous
import functools

import jax
import jax.numpy as jnp
from jax import lax
from jax.experimental import pallas as pl
from jax.experimental.pallas import tpu as pltpu
from jax.experimental.pallas import tpu_sc as plsc

H = 64
NE = 32
NA = 16
NC = 2
NS = 16
NW = NC * NS
CH = 128
NB = 4
NB2 = 2


def _table_body(eemb_ref, aemb_ref, w_ref, b_ref, t_ref):
    we = w_ref[:, :H]
    wa = w_ref[:, H:]
    dn = (((1,), (1,)), ((), ()))
    pe = lax.dot_general(eemb_ref[...], we, dn,
                         preferred_element_type=jnp.float32,
                         precision=lax.Precision.HIGHEST)
    pa = lax.dot_general(aemb_ref[...], wa, dn,
                         preferred_element_type=jnp.float32,
                         precision=lax.Precision.HIGHEST)
    t = pe[:, None, :] + pa[None, :, :] + b_ref[0][None, None, :]
    t_ref[...] = jnp.concatenate([t, jnp.zeros_like(t)], axis=-1)


def _idx_body(e_ref, a_ref, o_ref):
    o_ref[...] = e_ref[...] * NA + a_ref[...]


def _sc_gather_body(t_hbm, idx_hbm, out_hbm, idx_v, rows128, rows64, *sems):
    sems_g, sems_s = sems[:NB], sems[NB:]
    n_ch = idx_v.shape[0]
    wid = lax.axis_index("s") * NC + lax.axis_index("c")
    ch_base = wid * n_ch
    pltpu.sync_copy(idx_hbm.at[pl.ds(ch_base, n_ch)], idx_v)
    n_rounds = n_ch // NB

    def gather(j, s):
        return pltpu.make_async_copy(t_hbm.at[idx_v.at[j]], rows128.at[s],
                                     sems_g[s])

    def store(j, sc):
        return pltpu.make_async_copy(
            rows64.at[sc], out_hbm.at[pl.ds((ch_base + j) * CH, CH)],
            sems_s[sc])

    def compact(s, sc):
        def row(r, _):
            for c in range(H // 16):
                rows64[sc, r, pl.ds(c * 16, 16)] = (
                    rows128[s, r, pl.ds(c * 16, 16)])
            return 0

        lax.fori_loop(0, CH, row, 0)

    for s in range(NB):
        gather(s, s).start()

    def round_body(m, _):
        j0 = m * NB
        for s in range(NB):
            sc = s % NB2
            gather(j0 + s, s).wait()
            if s < NB2:
                @pl.when(m > 0)
                def _():
                    store(j0 + s - NB2, sc).wait()
            else:
                store(j0 + s - NB2, sc).wait()
            compact(s, sc)
            store(j0 + s, sc).start()

            @pl.when(m + 1 < n_rounds)
            def _():
                gather(j0 + NB + s, s).start()
        return 0

    lax.fori_loop(0, n_rounds, round_body, 0)
    for sc in range(NB2):
        store(n_ch - NB2 + sc, sc).wait()


def kernel(entity, attr, entity_emb, attr_emb, W, b):
    B, L = entity.shape
    n = B * L
    assert n % (NW * CH) == 0
    n_ch = n // (NW * CH)

    t3 = pl.pallas_call(
        _table_body,
        out_shape=jax.ShapeDtypeStruct((NE, NA, 2 * H), jnp.float32),
    )(entity_emb, attr_emb, W, b.reshape(1, H))
    table = t3.reshape(NE * NA, 2 * H)

    e2 = entity.reshape(n // CH, CH)
    a2 = attr.reshape(n // CH, CH)
    rows_per_blk = n // CH // 8
    idx = pl.pallas_call(
        _idx_body,
        grid=(8,),
        in_specs=[pl.BlockSpec((rows_per_blk, CH), lambda i: (i, 0))] * 2,
        out_specs=pl.BlockSpec((rows_per_blk, CH), lambda i: (i, 0)),
        out_shape=jax.ShapeDtypeStruct((n // CH, CH), jnp.int32),
    )(e2, a2)

    mesh = plsc.VectorSubcoreMesh(core_axis_name="c", subcore_axis_name="s",
                                  num_cores=NC, num_subcores=NS)
    gather = functools.partial(
        pl.kernel,
        out_type=jax.ShapeDtypeStruct((n, H), jnp.float32),
        mesh=mesh,
        compiler_params=pltpu.CompilerParams(use_tc_tiling_on_sc=True),
        scratch_types=(
            [pltpu.VMEM((n_ch, CH), jnp.int32),
             pltpu.VMEM((NB, CH, 2 * H), jnp.float32),
             pltpu.VMEM((NB2, CH, H), jnp.float32)]
            + [pltpu.SemaphoreType.DMA] * (NB + NB2)
        ),
    )(_sc_gather_body)
    return gather(table, idx).reshape(B, L, H)
